# Initial kernel scaffold; baseline (speedup 1.0000x reference)
import jax
import jax.numpy as jnp
from jax.experimental import pallas as pl


def kernel(x, edge_index, batch, emb, W_lin1, b_lin1, W_upd1, p1, W_lin2, b_lin2, W_upd2, p2, W_lin3, b_lin3, W_upd3, p3, W1, b1, W2, b2, W3, b3):
    raise NotImplementedError("write your pallas kernel here")


# speedup vs baseline: 1.1153x; 1.1153x over previous
import functools

import jax
import jax.numpy as jnp
from jax.experimental import pallas as pl
from jax.experimental.pallas import tpu as pltpu

N = 10000
E = 160000
D = 128


def _sage_fast(h, src, dst, valid, n, Wl, bl, Wu):
    y = jax.nn.relu(h @ Wl + bl)
    deff = jnp.where(valid, dst, n)
    aggr = jax.ops.segment_max(y[src], deff, num_segments=n + 1)[:n]
    aggr = jnp.maximum(aggr, y)
    return jax.nn.relu(jnp.concatenate([aggr, h], axis=1) @ Wu)


def _pool_fast(h, src, dst, valid, n, k, p):
    score = jnp.tanh((h @ p) / jnp.linalg.norm(p))
    vals, perm = jax.lax.top_k(score, k)
    hn = h[perm] * vals[:, None]
    inv = jnp.full((n,), -1, dtype=jnp.int32).at[perm].set(
        jnp.arange(k, dtype=jnp.int32))
    ns = inv[src]
    nd = inv[dst]
    nv = valid & (ns >= 0) & (nd >= 0)
    return hn, jnp.where(nv, ns, 0), jnp.where(nv, nd, 0), nv


def _readout(h):
    gmax = jnp.max(h, axis=0, keepdims=True)
    gmean = jnp.mean(h, axis=0, keepdims=True)
    return jnp.concatenate([gmax, gmean], axis=1)


def _mlp_body(z_ref, w1_ref, b1_ref, w2_ref, b2_ref, w3_ref, b3_ref, o_ref):
    z = z_ref[...]
    z = jax.nn.relu(
        jax.lax.dot_general(z, w1_ref[...], (((1,), (0,)), ((), ())),
                            preferred_element_type=jnp.float32) + b1_ref[...])
    z = jax.nn.relu(
        jax.lax.dot_general(z, w2_ref[...], (((1,), (0,)), ((), ())),
                            preferred_element_type=jnp.float32) + b2_ref[...])
    z = jax.lax.dot_general(z, w3_ref[...], (((1,), (0,)), ((), ())),
                            preferred_element_type=jnp.float32) + b3_ref[...]
    o_ref[...] = jax.nn.sigmoid(z)


def _mlp_head(z, W1, b1, W2, b2, W3, b3):
    return pl.pallas_call(
        _mlp_body,
        out_shape=jax.ShapeDtypeStruct((1, 1), jnp.float32),
    )(z, W1, b1[None], W2, b2[None], W3, b3[None])


def kernel(x, edge_index, batch, emb, W_lin1, b_lin1, W_upd1, p1, W_lin2,
           b_lin2, W_upd2, p2, W_lin3, b_lin3, W_upd3, p3, W1, b1, W2, b2,
           W3, b3):
    h = emb[x[:, 0]]
    src, dst = edge_index[0], edge_index[1]
    valid = jnp.ones((E,), dtype=bool)

    h = jax.nn.relu(_sage_fast(h, src, dst, valid, N, W_lin1, b_lin1, W_upd1))
    h, src, dst, valid = _pool_fast(h, src, dst, valid, N, 8000, p1)
    x1 = _readout(h)
    h = jax.nn.relu(_sage_fast(h, src, dst, valid, 8000, W_lin2, b_lin2,
                               W_upd2))
    h, src, dst, valid = _pool_fast(h, src, dst, valid, 8000, 6400, p2)
    x2 = _readout(h)
    h = jax.nn.relu(_sage_fast(h, src, dst, valid, 6400, W_lin3, b_lin3,
                               W_upd3))
    h, src, dst, valid = _pool_fast(h, src, dst, valid, 6400, 5120, p3)
    x3 = _readout(h)

    z = x1 + x2 + x3
    out = _mlp_head(z, W1, b1, W2, b2, W3, b3)
    return out[:, 0]


# speedup vs baseline: 1.4069x; 1.2615x over previous
import functools

import jax
import jax.numpy as jnp
from jax import lax
from jax.experimental import pallas as pl
from jax.experimental.pallas import tpu as pltpu
from jax.experimental.pallas import tpu_sc as plsc

N = 10000
E = 160000
D = 128


def _s0(v):
    return lax.squeeze(lax.slice(v, (0,), (1,)), (0,))


@functools.cache
def _make_segmax(n):
    assert n % 16 == 0
    RG = n // 16
    RGA = ((RG + 1 + 15) // 16) * 16
    EH = E // 2
    CE = 2000
    NCH = EH // CE
    assert NCH * CE == EH
    NV = CE // 16
    mesh = plsc.VectorSubcoreMesh(core_axis_name="c", subcore_axis_name="s")

    @functools.partial(
        pl.kernel,
        out_type=jax.ShapeDtypeStruct((2, RG, 16, 128), jnp.float32),
        mesh=mesh,
        compiler_params=pltpu.CompilerParams(needs_layout_passes=False),
        scratch_types=[
            pltpu.VMEM((RGA, 128), jnp.float32),
            pltpu.VMEM((CE,), jnp.int32),
            pltpu.VMEM((CE,), jnp.int32),
            pltpu.VMEM((CE,), jnp.int32),
            pltpu.VMEM((CE,), jnp.int32),
            pltpu.VMEM((CE + 32,), jnp.int32),
            pltpu.VMEM((CE + 32,), jnp.int32),
            pltpu.VMEM((16, 128), jnp.float32),
            pltpu.VMEM((16, 128), jnp.float32),
            pltpu.SemaphoreType.DMA,
            pltpu.SemaphoreType.DMA,
            pltpu.SemaphoreType.DMA,
            pltpu.SemaphoreType.DMA,
            pltpu.SemaphoreType.DMA,
            pltpu.SemaphoreType.DMA,
        ],
    )
    def segmax(y2d, src_h, dst_h, out_h, aggr, dbuf0, dbuf1, sbuf0,
               sbuf1, qs, qr, rows0, rows1, sem_d0, sem_d1, sem_s0, sem_s1,
               sem_r0, sem_r1):
        c = lax.axis_index("c")
        t = lax.axis_index("s")

        def self_idx(rg):
            return lax.iota(jnp.int32, 16) * 16 + (rg * 256 + t)

        for rg in range(RGA // 16):
            pltpu.async_copy(y2d.at[self_idx(rg)],
                             aggr.at[pl.ds(rg * 16, 16)], sem_r0)
        for rg in range(RGA // 16):
            pltpu.make_async_copy(y2d.at[self_idx(rg)],
                                  aggr.at[pl.ds(rg * 16, 16)],
                                  sem_r0).wait()

        ebase = c * EH

        def start(ch, dref, sref, semd, sems):
            base = ebase + ch * CE
            pltpu.async_copy(dst_h.at[pl.ds(base, CE)], dref, semd)
            pltpu.async_copy(src_h.at[pl.ds(base, CE)], sref, sems)

        def wait(ch, dref, sref, semd, sems):
            base = ebase + ch * CE
            pltpu.make_async_copy(dst_h.at[pl.ds(base, CE)], dref,
                                  semd).wait()
            pltpu.make_async_copy(src_h.at[pl.ds(base, CE)], sref,
                                  sems).wait()

        def process(dref, sref):
            def vec_body(i, qoff):
                dv = dref[pl.ds(i * 16, 16)]
                m = ((dv & 15) == t) & (dv < n)
                nh = _s0(plsc.all_reduce_population_count(m))

                def append():
                    sv = sref[pl.ds(i * 16, 16)]
                    dvs, svs, _m = plsc.sort_key_val(dv, sv, mask=m)
                    qs[pl.ds(qoff, 16)] = svs
                    qr[pl.ds(qoff, 16)] = dvs >> 4
                    return qoff + nh

                return lax.cond(nh > 0, append, lambda: qoff)

            qoff = lax.fori_loop(0, NV, vec_body, 0)
            qs[pl.ds(qoff, 16)] = jnp.full((16,), n, jnp.int32)
            qr[pl.ds(qoff, 16)] = jnp.full((16,), RG, jnp.int32)
            ngrp = (qoff + 15) >> 4

            def issue(g, rbuf, sem):
                pltpu.async_copy(y2d.at[qs.at[pl.ds(g * 16, 16)]], rbuf, sem)

            def drain(g, rbuf, sem):
                pltpu.make_async_copy(y2d.at[qs.at[pl.ds(g * 16, 16)]], rbuf,
                                      sem).wait()

            def process16(g, rbuf):
                rv = qr[pl.ds(g * 16, 16)]
                for j in range(16):
                    r = lax.squeeze(lax.slice(rv, (j,), (j + 1,)), (0,))
                    for kk in range(8):
                        s16 = pl.ds(kk * 16, 16)
                        aggr[r, s16] = jnp.maximum(aggr[r, s16],
                                                   rbuf[j, s16])

            @pl.when(ngrp > 0)
            def _():
                issue(0, rows0, sem_r0)

            def gpair(p, _):
                g0 = 2 * p
                g1 = 2 * p + 1

                @pl.when(g1 < ngrp)
                def _():
                    issue(g1, rows1, sem_r1)

                drain(g0, rows0, sem_r0)
                process16(g0, rows0)

                @pl.when(g1 + 1 < ngrp)
                def _():
                    issue(g1 + 1, rows0, sem_r0)

                @pl.when(g1 < ngrp)
                def _():
                    drain(g1, rows1, sem_r1)
                    process16(g1, rows1)

                return 0

            lax.fori_loop(0, (ngrp + 1) // 2, gpair, 0)

        start(0, dbuf0, sbuf0, sem_d0, sem_s0)

        def pair_body(p, _):
            ch0 = 2 * p
            ch1 = 2 * p + 1
            start(ch1, dbuf1, sbuf1, sem_d1, sem_s1)
            wait(ch0, dbuf0, sbuf0, sem_d0, sem_s0)
            process(dbuf0, sbuf0)

            @pl.when(ch1 + 1 < NCH)
            def _():
                start(ch1 + 1, dbuf0, sbuf0, sem_d0, sem_s0)

            wait(ch1, dbuf1, sbuf1, sem_d1, sem_s1)
            process(dbuf1, sbuf1)
            return 0

        assert NCH % 2 == 0
        lax.fori_loop(0, NCH // 2, pair_body, 0)
        pltpu.sync_copy(aggr.at[pl.ds(0, RG)], out_h.at[c, :, t])

    return segmax


def _segmax_sc(y, src, dst, n):
    RG = n // 16
    y2d = jnp.concatenate([y, jnp.zeros((256, 128), y.dtype)], axis=0)
    out = _make_segmax(n)(y2d, src, dst)
    return jnp.max(out, axis=0).reshape(n, 128)


def _sage_fast(h, src, dst, valid, n, Wl, bl, Wu):
    y = jax.nn.relu(h @ Wl + bl)
    deff = jnp.where(valid, dst, n)
    aggr = _segmax_sc(y, src, deff, n)
    return jax.nn.relu(jnp.concatenate([aggr, h], axis=1) @ Wu)


def _pool_fast(h, src, dst, valid, n, k, p):
    score = jnp.tanh((h @ p) / jnp.linalg.norm(p))
    vals, perm = jax.lax.top_k(score, k)
    hn = h[perm] * vals[:, None]
    inv = jnp.full((n,), -1, dtype=jnp.int32).at[perm].set(
        jnp.arange(k, dtype=jnp.int32))
    ns = inv[src]
    nd = inv[dst]
    nv = valid & (ns >= 0) & (nd >= 0)
    return hn, jnp.where(nv, ns, 0), jnp.where(nv, nd, 0), nv


def _readout(h):
    gmax = jnp.max(h, axis=0, keepdims=True)
    gmean = jnp.mean(h, axis=0, keepdims=True)
    return jnp.concatenate([gmax, gmean], axis=1)


def _mlp_body(z_ref, w1_ref, b1_ref, w2_ref, b2_ref, w3_ref, b3_ref, o_ref):
    z = z_ref[...]
    z = jax.nn.relu(
        jax.lax.dot_general(z, w1_ref[...], (((1,), (0,)), ((), ())),
                            preferred_element_type=jnp.float32) + b1_ref[...])
    z = jax.nn.relu(
        jax.lax.dot_general(z, w2_ref[...], (((1,), (0,)), ((), ())),
                            preferred_element_type=jnp.float32) + b2_ref[...])
    z = jax.lax.dot_general(z, w3_ref[...], (((1,), (0,)), ((), ())),
                            preferred_element_type=jnp.float32) + b3_ref[...]
    o_ref[...] = jax.nn.sigmoid(z)


def _mlp_head(z, W1, b1, W2, b2, W3, b3):
    return pl.pallas_call(
        _mlp_body,
        out_shape=jax.ShapeDtypeStruct((1, 1), jnp.float32),
    )(z, W1, b1[None], W2, b2[None], W3, b3[None])


def kernel(x, edge_index, batch, emb, W_lin1, b_lin1, W_upd1, p1, W_lin2,
           b_lin2, W_upd2, p2, W_lin3, b_lin3, W_upd3, p3, W1, b1, W2, b2,
           W3, b3):
    h = emb[x[:, 0]]
    src, dst = edge_index[0], edge_index[1]
    valid = jnp.ones((E,), dtype=bool)

    h = jax.nn.relu(_sage_fast(h, src, dst, valid, N, W_lin1, b_lin1, W_upd1))
    h, src, dst, valid = _pool_fast(h, src, dst, valid, N, 8000, p1)
    x1 = _readout(h)
    h = jax.nn.relu(_sage_fast(h, src, dst, valid, 8000, W_lin2, b_lin2,
                               W_upd2))
    h, src, dst, valid = _pool_fast(h, src, dst, valid, 8000, 6400, p2)
    x2 = _readout(h)
    h = jax.nn.relu(_sage_fast(h, src, dst, valid, 6400, W_lin3, b_lin3,
                               W_upd3))
    h, src, dst, valid = _pool_fast(h, src, dst, valid, 6400, 5120, p3)
    x3 = _readout(h)

    z = x1 + x2 + x3
    out = _mlp_head(z, W1, b1, W2, b2, W3, b3)
    return out[:, 0]
